# Initial kernel scaffold; baseline (speedup 1.0000x reference)
import jax
import jax.numpy as jnp
from jax.experimental import pallas as pl


def kernel(x, edge_index, W1, b1, W2, b2):
    raise NotImplementedError("write your pallas kernel here")


# speedup vs baseline: 19.1422x; 19.1422x over previous
import functools

import jax
import jax.numpy as jnp
from jax import lax
from jax.experimental import pallas as pl
from jax.experimental.pallas import tpu as pltpu
from jax.experimental.pallas import tpu_sc as plsc

_N = 10000
_NPAD = 10240
_NC = 2
_NS = 16
_NW = _NC * _NS
_RPT = _NPAD // _NS
_K = 80

_mesh = lambda: plsc.VectorSubcoreMesh(core_axis_name="c", subcore_axis_name="s")


def _fill_f32_1d(ref, n, value):
    v = jnp.full((16,), value, jnp.float32)

    @pl.loop(0, n // 16)
    def _(i):
        ref[pl.ds(i * 16, 16)] = v


def _fill_f32_2d(ref, rows, cols, value):
    v = jnp.full((16,), value, jnp.float32)

    @pl.loop(0, rows)
    def _(i):
        for c in range(cols // 16):
            ref[i, pl.ds(c * 16, 16)] = v


@functools.lru_cache()
def _make_count(nb):
    @functools.partial(
        pl.kernel,
        out_type=jax.ShapeDtypeStruct((_NC, _NPAD), jnp.float32),
        mesh=_mesh(),
        scratch_types=[
            pltpu.VMEM((nb, _K), jnp.int32),
            pltpu.VMEM((_K,), jnp.float32),
            pltpu.VMEM((_RPT,), jnp.float32),
            pltpu.VMEM_SHARED((_NPAD,), jnp.float32),
        ],
    )
    def count(dst_hbm, out_hbm, dst_v, ones_v, zb_v, acc):
        cid = lax.axis_index("c")
        sid = lax.axis_index("s")
        wid = cid * _NS + sid
        _fill_f32_1d(ones_v, _K, 1.0)
        _fill_f32_1d(zb_v, _RPT, 0.0)
        pltpu.sync_copy(zb_v, acc.at[pl.ds(sid * _RPT, _RPT)])
        plsc.subcore_barrier()
        pltpu.sync_copy(dst_hbm.at[wid], dst_v)

        @pl.loop(0, nb)
        def _(j):
            pltpu.sync_copy(ones_v, acc.at[dst_v.at[j]], add=True)

        plsc.subcore_barrier()
        sl = pl.ds(sid * _RPT, _RPT)
        pltpu.sync_copy(acc.at[sl], out_hbm.at[cid, sl])

    return count


@functools.lru_cache()
def _make_agg(nb, d):
    @functools.partial(
        pl.kernel,
        out_type=jax.ShapeDtypeStruct((_NC, _NPAD, d), jnp.float32),
        mesh=_mesh(),
        scratch_types=[
            pltpu.VMEM((nb, _K), jnp.int32),
            pltpu.VMEM((nb, _K), jnp.int32),
            pltpu.VMEM((_K, d), jnp.float32),
            pltpu.VMEM_SHARED((_NPAD, d), jnp.float32),
            pltpu.SemaphoreType.DMA,
        ],
    )
    def agg(g_hbm, src_hbm, dst_hbm, out_hbm, src_v, dst_v, rows_v, acc, sem):
        cid = lax.axis_index("c")
        sid = lax.axis_index("s")
        wid = cid * _NS + sid
        _fill_f32_2d(rows_v, _K, d, 0.0)
        for t in range(_RPT // _K):
            pltpu.sync_copy(rows_v, acc.at[pl.ds(sid * _RPT + t * _K, _K)])
        plsc.subcore_barrier()
        pltpu.sync_copy(src_hbm.at[wid], src_v)
        pltpu.sync_copy(dst_hbm.at[wid], dst_v)

        @pl.loop(0, nb)
        def _(j):
            pltpu.async_copy(g_hbm.at[src_v.at[j]], rows_v, sem).wait()
            pltpu.sync_copy(rows_v, acc.at[dst_v.at[j]], add=True)

        plsc.subcore_barrier()
        sl = pl.ds(sid * _RPT, _RPT)
        pltpu.sync_copy(acc.at[sl], out_hbm.at[cid, sl])

    return agg


def _tc_layer1(x, w1, cnt0, cnt1, r=1000):
    n, din = x.shape
    dh = w1.shape[1]

    def body(x_ref, w_ref, c0_ref, c1_ref, g_ref, dis_ref):
        deg = c0_ref[...] + c1_ref[...] + 1.0
        dis = lax.rsqrt(deg)
        h = jnp.dot(x_ref[...], w_ref[...], preferred_element_type=jnp.float32)
        g_ref[...] = h * dis
        dis_ref[...] = dis

    return pl.pallas_call(
        body,
        grid=(n // r,),
        in_specs=[
            pl.BlockSpec((r, din), lambda i: (i, 0)),
            pl.BlockSpec((din, dh), lambda i: (0, 0)),
            pl.BlockSpec((r, 1), lambda i: (i, 0)),
            pl.BlockSpec((r, 1), lambda i: (i, 0)),
        ],
        out_specs=[
            pl.BlockSpec((r, dh), lambda i: (i, 0)),
            pl.BlockSpec((r, 1), lambda i: (i, 0)),
        ],
        out_shape=[
            jax.ShapeDtypeStruct((n, dh), jnp.float32),
            jax.ShapeDtypeStruct((n, 1), jnp.float32),
        ],
    )(x, w1, cnt0, cnt1)


def _tc_layer2(p0, p1, g1, dis, b1, r=1000):
    n, dh = g1.shape

    def body(p0_ref, p1_ref, g1_ref, dis_ref, b_ref, q_ref):
        dis = dis_ref[...]
        s = dis * (p0_ref[...] + p1_ref[...] + g1_ref[...]) + b_ref[...]
        q_ref[...] = jnp.maximum(s, 0.0) * dis

    return pl.pallas_call(
        body,
        grid=(n // r,),
        in_specs=[
            pl.BlockSpec((r, dh), lambda i: (i, 0)),
            pl.BlockSpec((r, dh), lambda i: (i, 0)),
            pl.BlockSpec((r, dh), lambda i: (i, 0)),
            pl.BlockSpec((r, 1), lambda i: (i, 0)),
            pl.BlockSpec((1, dh), lambda i: (0, 0)),
        ],
        out_specs=pl.BlockSpec((r, dh), lambda i: (i, 0)),
        out_shape=jax.ShapeDtypeStruct((n, dh), jnp.float32),
    )(p0, p1, g1, dis, b1)


def _tc_final(p0, p1, q, dis, w2, b2, r=1000):
    n, dh = q.shape
    do = w2.shape[1]

    def body(p0_ref, p1_ref, q_ref, dis_ref, w_ref, b_ref, o_ref):
        t = dis_ref[...] * (p0_ref[...] + p1_ref[...] + q_ref[...])
        o = jnp.dot(t, w_ref[...], preferred_element_type=jnp.float32)
        o = o + b_ref[...]
        m = jnp.max(o, axis=-1, keepdims=True)
        sh = o - m
        lse = jnp.log(jnp.sum(jnp.exp(sh), axis=-1, keepdims=True))
        o_ref[...] = sh - lse

    return pl.pallas_call(
        body,
        grid=(n // r,),
        in_specs=[
            pl.BlockSpec((r, dh), lambda i: (i, 0)),
            pl.BlockSpec((r, dh), lambda i: (i, 0)),
            pl.BlockSpec((r, dh), lambda i: (i, 0)),
            pl.BlockSpec((r, 1), lambda i: (i, 0)),
            pl.BlockSpec((dh, do), lambda i: (0, 0)),
            pl.BlockSpec((1, do), lambda i: (0, 0)),
        ],
        out_specs=pl.BlockSpec((r, do), lambda i: (i, 0)),
        out_shape=jax.ShapeDtypeStruct((n, do), jnp.float32),
    )(p0, p1, q, dis, w2, b2)


def kernel(x, edge_index, W1, b1, W2, b2):
    n = x.shape[0]
    e = edge_index.shape[1]
    assert n == _N and e % (_NW * _K) == 0
    nb = e // (_NW * _K)

    ei = edge_index.astype(jnp.int32)
    src = ei[0].reshape(_NW, nb, _K)
    dst = ei[1].reshape(_NW, nb, _K)

    cnt = _make_count(nb)(dst)
    cnt0 = cnt[0, :_N].reshape(_N, 1)
    cnt1 = cnt[1, :_N].reshape(_N, 1)

    g1, dis = _tc_layer1(x, W1, cnt0, cnt1)
    a1 = _make_agg(nb, g1.shape[1])(g1, src, dst)
    q = _tc_layer2(a1[0, :_N], a1[1, :_N], g1, dis,
                   b1.reshape(1, -1))
    a2 = _make_agg(nb, q.shape[1])(q, src, dst)
    return _tc_final(a2[0, :_N], a2[1, :_N], q, dis,
                     W2, b2.reshape(1, -1))
